# baseline (speedup 1.0000x reference)
import jax
import jax.numpy as jnp
from jax.experimental import pallas as pl
from jax.experimental.pallas import tpu as pltpu

_N = 10000
_F = 128
_BM = 200


def _s1_kernel(x_ref, w1_ref, s1_ref):
    s1_ref[...] = jnp.dot(
        x_ref[...], w1_ref[...],
        preferred_element_type=jnp.float32,
        precision=jax.lax.Precision.HIGHEST,
    ).astype(jnp.bfloat16)


def _layer1_kernel(s1_ref, b1_ref, w2_ref, g_ref, s2_ref):
    acc = jnp.dot(
        g_ref[...].astype(jnp.bfloat16), s1_ref[...],
        preferred_element_type=jnp.float32,
    )
    h = jnp.maximum(acc + b1_ref[...], 0.0)
    s2_ref[...] = jnp.dot(
        h, w2_ref[...],
        preferred_element_type=jnp.float32,
        precision=jax.lax.Precision.HIGHEST,
    ).astype(jnp.bfloat16)


def _layer2_kernel(s2_ref, b2_ref, g_ref, out_ref):
    acc = jnp.dot(
        g_ref[...].astype(jnp.bfloat16), s2_ref[...],
        preferred_element_type=jnp.float32,
    )
    out_ref[...] = acc + b2_ref[...]


def kernel(g, x, W1, b1, W2, b2):
    n_blocks = _N // _BM

    s1 = pl.pallas_call(
        _s1_kernel,
        out_shape=jax.ShapeDtypeStruct((_N, _F), jnp.bfloat16),
    )(x, W1)

    s2 = pl.pallas_call(
        _layer1_kernel,
        grid=(n_blocks,),
        in_specs=[
            pl.BlockSpec((_N, _F), lambda i: (0, 0)),
            pl.BlockSpec((1, _F), lambda i: (0, 0)),
            pl.BlockSpec((_F, _F), lambda i: (0, 0)),
            pl.BlockSpec((_BM, _N), lambda i: (i, 0)),
        ],
        out_specs=pl.BlockSpec((_BM, _F), lambda i: (i, 0)),
        out_shape=jax.ShapeDtypeStruct((_N, _F), jnp.bfloat16),
        compiler_params=pltpu.CompilerParams(
            dimension_semantics=("parallel",),
        ),
    )(s1, b1.reshape(1, _F), W2, g)

    out = pl.pallas_call(
        _layer2_kernel,
        grid=(n_blocks,),
        in_specs=[
            pl.BlockSpec((_N, _F), lambda i: (0, 0)),
            pl.BlockSpec((1, _F), lambda i: (0, 0)),
            pl.BlockSpec((_BM, _N), lambda i: (i, 0)),
        ],
        out_specs=pl.BlockSpec((_BM, _F), lambda i: (i, 0)),
        out_shape=jax.ShapeDtypeStruct((_N, _F), jnp.float32),
        compiler_params=pltpu.CompilerParams(
            dimension_semantics=("parallel",),
        ),
    )(s2, b2.reshape(1, _F), g)

    return out
